# baseline (device time: 121676 ns/iter reference)
import jax
import jax.numpy as jnp
from jax import lax
from jax.experimental import pallas as pl
from jax.experimental.pallas import tpu as pltpu

N_DEV = 8
B_LOC = 2
SQ = 128
HQ = 32
DH = 64
D_MODEL = 512
D_BLK = 256


def kernel(x, Wq, K_ext, V_ext, Wo):
    my = lax.axis_index("i")
    K_loc = lax.dynamic_slice_in_dim(K_ext, my * B_LOC, B_LOC, axis=0)
    V_loc = lax.dynamic_slice_in_dim(V_ext, my * B_LOC, B_LOC, axis=0)
    K_loc = jnp.transpose(K_loc, (0, 2, 1, 3))
    V_loc = jnp.transpose(V_loc, (0, 2, 1, 3))

    def body(x_ref, wq_ref, k_ref, v_ref, wo_ref, out_ref,
             wqg, wog, qbuf, ctxbuf,
             copy_sems, wq_send, wq_recv, wo_send, wo_recv):
        my_i = lax.axis_index("i")
        left = (my_i - 1) % N_DEV
        right = (my_i + 1) % N_DEV

        cq = pltpu.make_async_copy(wq_ref, wqg.at[my_i], copy_sems.at[0])
        co = pltpu.make_async_copy(wo_ref, wog.at[my_i], copy_sems.at[1])
        cq.start()
        co.start()
        cq.wait()
        co.wait()

        barrier_sem = pltpu.get_barrier_semaphore()
        for nbr in (left, right):
            pl.semaphore_signal(barrier_sem, inc=1, device_id=(nbr,),
                                device_id_type=pl.DeviceIdType.MESH)
        pl.semaphore_wait(barrier_sem, 2)

        for h in range(N_DEV - 1):
            o = (my_i - h) % N_DEV
            rq = pltpu.make_async_remote_copy(
                src_ref=wqg.at[o], dst_ref=wqg.at[o],
                send_sem=wq_send.at[h], recv_sem=wq_recv.at[h],
                device_id=(right,), device_id_type=pl.DeviceIdType.MESH)
            ro = pltpu.make_async_remote_copy(
                src_ref=wog.at[o], dst_ref=wog.at[o],
                send_sem=wo_send.at[h], recv_sem=wo_recv.at[h],
                device_id=(right,), device_id_type=pl.DeviceIdType.MESH)
            rq.start()
            ro.start()
            rq.wait()
            ro.wait()

        x2d = x_ref[...].reshape(B_LOC * SQ, D_MODEL)
        for o in range(N_DEV):
            qbuf[:, o * D_BLK:(o + 1) * D_BLK] = jnp.dot(
                x2d, wqg[o], preferred_element_type=jnp.float32)

        for b in range(B_LOC):
            for g in range(HQ):
                qh = qbuf[b * SQ:(b + 1) * SQ, g * DH:(g + 1) * DH]
                s = lax.dot_general(
                    qh, k_ref[b, g], (((1,), (1,)), ((), ())),
                    preferred_element_type=jnp.float32) * 0.125
                m = jnp.max(s, axis=-1, keepdims=True)
                e = jnp.exp(s - m)
                p = e / jnp.sum(e, axis=-1, keepdims=True)
                ctxbuf[b * SQ:(b + 1) * SQ, g * DH:(g + 1) * DH] = jnp.dot(
                    p, v_ref[b, g], preferred_element_type=jnp.float32)

        acc = jnp.zeros((B_LOC * SQ, D_MODEL), jnp.float32)
        for o in range(N_DEV):
            acc = acc + jnp.dot(
                ctxbuf[:, o * D_BLK:(o + 1) * D_BLK], wog[o],
                preferred_element_type=jnp.float32)
        out_ref[...] = acc.reshape(B_LOC, SQ, D_MODEL)

    return pl.pallas_call(
        body,
        out_shape=jax.ShapeDtypeStruct((B_LOC, SQ, D_MODEL), jnp.float32),
        in_specs=[pl.BlockSpec(memory_space=pltpu.VMEM)] * 5,
        out_specs=pl.BlockSpec(memory_space=pltpu.VMEM),
        scratch_shapes=[
            pltpu.VMEM((N_DEV, D_MODEL, D_BLK), jnp.float32),
            pltpu.VMEM((N_DEV, D_BLK, D_MODEL), jnp.float32),
            pltpu.VMEM((B_LOC * SQ, HQ * DH), jnp.float32),
            pltpu.VMEM((B_LOC * SQ, HQ * DH), jnp.float32),
            pltpu.SemaphoreType.DMA((2,)),
            pltpu.SemaphoreType.DMA((N_DEV - 1,)),
            pltpu.SemaphoreType.DMA((N_DEV - 1,)),
            pltpu.SemaphoreType.DMA((N_DEV - 1,)),
            pltpu.SemaphoreType.DMA((N_DEV - 1,)),
        ],
        compiler_params=pltpu.CompilerParams(collective_id=0),
    )(x, Wq, K_loc, V_loc, Wo)


# device time: 44150 ns/iter; 2.7560x vs baseline; 2.7560x over previous
import jax
import jax.numpy as jnp
from jax import lax
from jax.experimental import pallas as pl
from jax.experimental.pallas import tpu as pltpu

N_DEV = 8
B_LOC = 2
SQ = 128
HQ = 32
H_BLK = 4
DH = 64
D_MODEL = 512
D_BLK = H_BLK * DH
R_HOPS = 4
L_HOPS = 3
PACK = 2 * D_MODEL


def kernel(x, Wq, K_ext, V_ext, Wo):
    my = lax.axis_index("i")

    wc = jnp.concatenate([Wq, Wo.T], axis=0).astype(jnp.bfloat16)

    K_loc = lax.dynamic_slice_in_dim(K_ext, my * B_LOC, B_LOC, axis=0)
    V_loc = lax.dynamic_slice_in_dim(V_ext, my * B_LOC, B_LOC, axis=0)
    K_loc = jnp.transpose(K_loc, (0, 2, 1, 3)).astype(jnp.bfloat16)
    V_loc = jnp.transpose(V_loc, (0, 2, 1, 3)).astype(jnp.bfloat16)
    order = jnp.mod(my + jnp.array([0, -1, -2, -3, -4, 1, 2, 3]), N_DEV)
    heads = (order[:, None] * H_BLK + jnp.arange(H_BLK)).reshape(-1)
    K_loc = jnp.take(K_loc, heads, axis=1)
    V_loc = jnp.take(V_loc, heads, axis=1)

    def body(x_ref, wc_ref, k_ref, v_ref, out_ref,
             wcg, r_send, r_recv, l_send, l_recv):
        my_i = lax.axis_index("i")
        left = (my_i - 1) % N_DEV
        right = (my_i + 1) % N_DEV

        barrier_sem = pltpu.get_barrier_semaphore()
        for nbr in (left, right):
            pl.semaphore_signal(barrier_sem, inc=1, device_id=(nbr,),
                                device_id_type=pl.DeviceIdType.MESH)
        pl.semaphore_wait(barrier_sem, 2)

        rd = [
            pltpu.make_async_remote_copy(
                src_ref=wc_ref if h == 0 else wcg.at[h - 1],
                dst_ref=wcg.at[h],
                send_sem=r_send.at[h], recv_sem=r_recv.at[h],
                device_id=(right,), device_id_type=pl.DeviceIdType.MESH)
            for h in range(R_HOPS)
        ]
        ld = [
            pltpu.make_async_remote_copy(
                src_ref=wc_ref if h == 0 else wcg.at[R_HOPS + h - 1],
                dst_ref=wcg.at[R_HOPS + h],
                send_sem=l_send.at[h], recv_sem=l_recv.at[h],
                device_id=(left,), device_id_type=pl.DeviceIdType.MESH)
            for h in range(L_HOPS)
        ]

        xb = x_ref[...].reshape(B_LOC * SQ, D_MODEL).astype(jnp.bfloat16)

        def contribution(w_chunk, kv_blk):
            q = jnp.dot(xb, w_chunk[:D_MODEL, :],
                        preferred_element_type=jnp.float32)
            q = (q * 0.125).astype(jnp.bfloat16)
            rows = []
            for b in range(B_LOC):
                ctx_h = []
                for hh in range(H_BLK):
                    qh = q[b * SQ:(b + 1) * SQ, hh * DH:(hh + 1) * DH]
                    g = kv_blk * H_BLK + hh
                    s = lax.dot_general(
                        qh, k_ref[b, g], (((1,), (1,)), ((), ())),
                        preferred_element_type=jnp.float32)
                    m = jnp.max(s, axis=-1, keepdims=True)
                    e = jnp.exp(s - m)
                    p = (e / jnp.sum(e, axis=-1, keepdims=True)).astype(
                        jnp.bfloat16)
                    ctx_h.append(jnp.dot(p, v_ref[b, g],
                                         preferred_element_type=jnp.float32))
                rows.append(jnp.concatenate(ctx_h, axis=1))
            ctx = jnp.concatenate(rows, axis=0).astype(jnp.bfloat16)
            return lax.dot_general(
                ctx, w_chunk[D_MODEL:, :], (((1,), (1,)), ((), ())),
                preferred_element_type=jnp.float32)

        rd[0].start()
        ld[0].start()
        acc = contribution(wc_ref[...], 0)

        for h in range(R_HOPS):
            rd[h].wait_recv()
            if h + 1 < R_HOPS:
                rd[h + 1].start()
            if h < L_HOPS:
                ld[h].wait_recv()
                if h + 1 < L_HOPS:
                    ld[h + 1].start()
            acc = acc + contribution(wcg[h], 1 + h)
            if h < L_HOPS:
                acc = acc + contribution(wcg[R_HOPS + h], 1 + R_HOPS + h)

        for d in rd + ld:
            d.wait_send()

        out_ref[...] = acc.reshape(B_LOC, SQ, D_MODEL)

    return pl.pallas_call(
        body,
        out_shape=jax.ShapeDtypeStruct((B_LOC, SQ, D_MODEL), jnp.float32),
        in_specs=[pl.BlockSpec(memory_space=pltpu.VMEM)] * 4,
        out_specs=pl.BlockSpec(memory_space=pltpu.VMEM),
        scratch_shapes=[
            pltpu.VMEM((R_HOPS + L_HOPS, PACK, D_BLK), jnp.bfloat16),
            pltpu.SemaphoreType.DMA((R_HOPS,)),
            pltpu.SemaphoreType.DMA((R_HOPS,)),
            pltpu.SemaphoreType.DMA((L_HOPS,)),
            pltpu.SemaphoreType.DMA((L_HOPS,)),
        ],
        compiler_params=pltpu.CompilerParams(collective_id=0),
    )(x, wc, K_loc, V_loc)


# device time: 37166 ns/iter; 3.2739x vs baseline; 1.1879x over previous
import jax
import jax.numpy as jnp
from jax import lax
from jax.experimental import pallas as pl
from jax.experimental.pallas import tpu as pltpu

N_DEV = 8
B_LOC = 2
SQ = 128
HQ = 32
H_BLK = 4
DH = 64
D_MODEL = 512
D_BLK = H_BLK * DH

_ARRIVAL_MASKS = [0, 1, 3, 4, 5, 2, 7, 6]


def kernel(x, Wq, K_ext, V_ext, Wo):
    my = lax.axis_index("i")

    wc = jnp.stack([Wq, Wo.T]).astype(jnp.bfloat16)

    K_loc = lax.dynamic_slice_in_dim(K_ext, my * B_LOC, B_LOC, axis=0)
    V_loc = lax.dynamic_slice_in_dim(V_ext, my * B_LOC, B_LOC, axis=0)
    K_loc = jnp.transpose(K_loc, (0, 2, 1, 3)).astype(jnp.bfloat16)
    V_loc = jnp.transpose(V_loc, (0, 2, 1, 3)).astype(jnp.bfloat16)
    order = my ^ jnp.array(_ARRIVAL_MASKS)
    heads = (order[:, None] * H_BLK + jnp.arange(H_BLK)).reshape(-1)
    K_loc = jnp.take(K_loc, heads, axis=1)
    V_loc = jnp.take(V_loc, heads, axis=1)

    def body(x_ref, wc_ref, k_ref, v_ref, out_ref,
             wcg, xs, xr, ys, yr, zs, zr):
        my_i = lax.axis_index("i")
        nx = my_i ^ 1
        ny = my_i ^ 3
        nz = my_i ^ 4

        barrier_sem = pltpu.get_barrier_semaphore()
        for nbr in (nx, ny, nz):
            pl.semaphore_signal(barrier_sem, inc=1, device_id=(nbr,),
                                device_id_type=pl.DeviceIdType.MESH)
        pl.semaphore_wait(barrier_sem, 3)

        def rc(src, dst, send_sem, recv_sem, dev):
            return pltpu.make_async_remote_copy(
                src_ref=src, dst_ref=dst, send_sem=send_sem,
                recv_sem=recv_sem, device_id=(dev,),
                device_id_type=pl.DeviceIdType.MESH)

        r0x = rc(wc_ref, wcg.at[0], xs.at[0], xr.at[0], nx)
        r0y = rc(wc_ref, wcg.at[1], ys.at[0], yr.at[0], ny)
        r0z = rc(wc_ref, wcg.at[2], zs.at[0], zr.at[0], nz)
        r1x = rc(wcg.at[2], wcg.at[3], xs.at[1], xr.at[1], nx)
        r1y = rc(wcg.at[0], wcg.at[4], ys.at[1], yr.at[1], ny)
        r1z = rc(wcg.at[1], wcg.at[5], zs.at[1], zr.at[1], nz)
        r2x = rc(wcg.at[5, 0], wcg.at[6, 0], xs.at[2], xr.at[2], nx)
        r2y = rc(wcg.at[3, 1], wcg.at[6, 1], ys.at[2], yr.at[2], ny)

        xb = x_ref[...].reshape(B_LOC * SQ, D_MODEL).astype(jnp.bfloat16)

        def contribution(wq_p, wot_p, blk):
            q = jnp.dot(xb, wq_p, preferred_element_type=jnp.float32)
            q = (q * 0.125).astype(jnp.bfloat16)
            rows = []
            for b in range(B_LOC):
                ctx_h = []
                for hh in range(H_BLK):
                    qh = q[b * SQ:(b + 1) * SQ, hh * DH:(hh + 1) * DH]
                    g = blk * H_BLK + hh
                    s = lax.dot_general(
                        qh, k_ref[b, g], (((1,), (1,)), ((), ())),
                        preferred_element_type=jnp.float32)
                    m = jnp.max(s, axis=-1, keepdims=True)
                    e = jnp.exp(s - m)
                    p = (e / jnp.sum(e, axis=-1, keepdims=True)).astype(
                        jnp.bfloat16)
                    ctx_h.append(jnp.dot(p, v_ref[b, g],
                                         preferred_element_type=jnp.float32))
                rows.append(jnp.concatenate(ctx_h, axis=1))
            ctx = jnp.concatenate(rows, axis=0).astype(jnp.bfloat16)
            return lax.dot_general(
                ctx, wot_p, (((1,), (1,)), ((), ())),
                preferred_element_type=jnp.float32)

        slot_c = lambda s: contribution(wcg[s, 0], wcg[s, 1], 1 + s)

        r0x.start()
        r0y.start()
        r0z.start()
        acc = contribution(wc_ref[0], wc_ref[1], 0)

        r0x.wait_recv()
        r1y.start()
        r0y.wait_recv()
        r1z.start()
        r0z.wait_recv()
        r1x.start()
        acc = acc + slot_c(0) + slot_c(1) + slot_c(2)

        r1z.wait_recv()
        r2x.start()
        r1x.wait_recv()
        r2y.start()
        r1y.wait_recv()
        acc = acc + slot_c(3) + slot_c(4) + slot_c(5)

        r2x.wait_recv()
        r2y.wait_recv()
        acc = acc + slot_c(6)

        for d in (r0x, r0y, r0z, r1x, r1y, r1z, r2x, r2y):
            d.wait_send()

        out_ref[...] = acc.reshape(B_LOC, SQ, D_MODEL)

    return pl.pallas_call(
        body,
        out_shape=jax.ShapeDtypeStruct((B_LOC, SQ, D_MODEL), jnp.float32),
        in_specs=[pl.BlockSpec(memory_space=pltpu.VMEM)] * 4,
        out_specs=pl.BlockSpec(memory_space=pltpu.VMEM),
        scratch_shapes=[
            pltpu.VMEM((7, 2, D_MODEL, D_BLK), jnp.bfloat16),
            pltpu.SemaphoreType.DMA((3,)),
            pltpu.SemaphoreType.DMA((3,)),
            pltpu.SemaphoreType.DMA((3,)),
            pltpu.SemaphoreType.DMA((3,)),
            pltpu.SemaphoreType.DMA((2,)),
            pltpu.SemaphoreType.DMA((2,)),
        ],
        compiler_params=pltpu.CompilerParams(collective_id=0),
    )(x, wc, K_loc, V_loc)
